# Initial kernel scaffold; baseline (speedup 1.0000x reference)
import jax
import jax.numpy as jnp
from jax.experimental import pallas as pl


def kernel(x, edge_index, W1, b1, W2, b2):
    raise NotImplementedError("write your pallas kernel here")


# speedup vs baseline: 10.4857x; 10.4857x over previous
import functools

import jax
import jax.numpy as jnp
from jax import lax
from jax.experimental import pallas as pl
from jax.experimental.pallas import tpu as pltpu
from jax.experimental.pallas import tpu_sc as plsc

N = 10000
D = 256
E = 160000
BM = 512
NP = 10240
NCORE = 2
NTILE = 16
RPT = NP // NTILE
CHUNK = 128
E_PAD = 163840
NCH_DEG = E_PAD // (NCORE * NTILE * CHUNK)
NCH_PROP = E_PAD // (NTILE * CHUNK)
HALF = D // 2


@functools.cache
def _sc_mesh():
    return plsc.VectorSubcoreMesh(
        core_axis_name="c", subcore_axis_name="s",
        num_cores=NCORE, num_subcores=NTILE)


def _deg_body(src_hbm, dst_hbm, z1_hbm, cnt_hbm, loop_hbm,
              srcv, dstv, onesv, lbuf, cnt_sh, loop_sh):
    c = lax.axis_index("c")
    s = lax.axis_index("s")
    t = c * NTILE + s
    pltpu.sync_copy(src_hbm.at[t], srcv)
    pltpu.sync_copy(dst_hbm.at[t], dstv)
    r0 = s * RPT
    pltpu.sync_copy(z1_hbm.at[pl.ds(r0, RPT)], cnt_sh.at[pl.ds(r0, RPT)])
    pltpu.sync_copy(z1_hbm.at[pl.ds(r0, RPT)], loop_sh.at[pl.ds(r0, RPT)])
    for k in range(CHUNK // 16):
        onesv[pl.ds(k * 16, 16)] = jnp.full((16,), 1.0, jnp.float32)
    plsc.subcore_barrier()

    @pl.loop(0, NCH_DEG)
    def _chunk(j):
        for k in range(CHUNK // 16):
            sv = srcv[j, pl.ds(k * 16, 16)]
            dv = dstv[j, pl.ds(k * 16, 16)]
            lbuf[pl.ds(k * 16, 16)] = jnp.where(sv == dv, 1.0, 0.0)
        pltpu.sync_copy(onesv, cnt_sh.at[dstv.at[j]], add=True)
        pltpu.sync_copy(lbuf, loop_sh.at[dstv.at[j]], add=True)

    plsc.subcore_barrier()
    pltpu.sync_copy(cnt_sh.at[pl.ds(r0, RPT)], cnt_hbm.at[c, pl.ds(r0, RPT)])
    pltpu.sync_copy(loop_sh.at[pl.ds(r0, RPT)], loop_hbm.at[c, pl.ds(r0, RPT)])


@functools.cache
def _deg_call():
    return pl.kernel(
        _deg_body,
        out_type=[jax.ShapeDtypeStruct((NCORE, NP), jnp.float32),
                  jax.ShapeDtypeStruct((NCORE, NP), jnp.float32)],
        mesh=_sc_mesh(),
        scratch_types=[
            pltpu.VMEM((NCH_DEG, CHUNK), jnp.int32),
            pltpu.VMEM((NCH_DEG, CHUNK), jnp.int32),
            pltpu.VMEM((CHUNK,), jnp.float32),
            pltpu.VMEM((CHUNK,), jnp.float32),
            pltpu.VMEM_SHARED((NP,), jnp.float32),
            pltpu.VMEM_SHARED((NP,), jnp.float32),
        ],
    )


def _prop_body(ycat_hbm, src_hbm, dst_hbm, z2_hbm, out_hbm,
               idxv, dstv, buf, acc_sh):
    c = lax.axis_index("c")
    s = lax.axis_index("s")
    pltpu.sync_copy(src_hbm.at[c, s], idxv)
    pltpu.sync_copy(dst_hbm.at[s], dstv)
    r0 = s * RPT
    pltpu.sync_copy(z2_hbm.at[pl.ds(r0, RPT)], acc_sh.at[pl.ds(r0, RPT)])
    plsc.subcore_barrier()

    @pl.loop(0, NCH_PROP)
    def _chunk(j):
        pltpu.sync_copy(ycat_hbm.at[idxv.at[j]], buf)
        pltpu.sync_copy(buf, acc_sh.at[dstv.at[j]], add=True)

    plsc.subcore_barrier()
    pltpu.sync_copy(acc_sh.at[pl.ds(r0, RPT)], out_hbm.at[c, pl.ds(r0, RPT)])


@functools.cache
def _prop_call():
    return pl.kernel(
        _prop_body,
        out_type=jax.ShapeDtypeStruct((NCORE, NP, HALF), jnp.float32),
        mesh=_sc_mesh(),
        scratch_types=[
            pltpu.VMEM((NCH_PROP, CHUNK), jnp.int32),
            pltpu.VMEM((NCH_PROP, CHUNK), jnp.int32),
            pltpu.VMEM((CHUNK, HALF), jnp.float32),
            pltpu.VMEM_SHARED((NP, HALF), jnp.float32),
        ],
    )


def _norm(cnt_ref, loop_ref):
    cnt = cnt_ref[0] + cnt_ref[1]
    lc = loop_ref[0] + loop_ref[1]
    wl = jnp.where(lc == 0.0, 1.0, 0.0)
    d = lax.rsqrt(cnt + wl)
    return d, d * d * wl


def _scale1_body(x_ref, w_ref, cnt_ref, loop_ref, y_ref, xw_ref):
    xw = jnp.dot(x_ref[...], w_ref[...], preferred_element_type=jnp.float32)
    d, _ = _norm(cnt_ref, loop_ref)
    y = xw * d
    xw_ref[...] = xw
    y_ref[0] = y[:, :HALF]
    y_ref[1] = y[:, HALF:]


def _mid_body(acc_ref, xw_ref, cnt_ref, loop_ref, b_ref, w2_ref,
              y_ref, hw_ref):
    d, dw = _norm(cnt_ref, loop_ref)
    agg = jnp.concatenate([acc_ref[0], acc_ref[1]], axis=1)
    h = agg * d + xw_ref[...] * dw + b_ref[...]
    h = jnp.maximum(h, 0.0)
    hw = jnp.dot(h, w2_ref[...], preferred_element_type=jnp.float32)
    hw_ref[...] = hw
    y = hw * d
    y_ref[0] = y[:, :HALF]
    y_ref[1] = y[:, HALF:]


def _final_body(acc_ref, hw_ref, cnt_ref, loop_ref, b_ref, out_ref):
    d, dw = _norm(cnt_ref, loop_ref)
    agg = jnp.concatenate([acc_ref[0], acc_ref[1]], axis=1)
    o = agg * d + hw_ref[...] * dw + b_ref[...]
    m = jnp.max(o, axis=1, keepdims=True)
    e = jnp.exp(o - m)
    out_ref[...] = e / jnp.sum(e, axis=1, keepdims=True)


_spec_rows = pl.BlockSpec((BM, D), lambda i: (i, 0))
_spec_w = pl.BlockSpec((D, D), lambda i: (0, 0))
_spec_nrm = pl.BlockSpec((2, BM, 1), lambda i: (0, i, 0))
_spec_cat = pl.BlockSpec((2, BM, HALF), lambda i: (0, i, 0))
_spec_b = pl.BlockSpec((1, D), lambda i: (0, 0))

_scale1 = pl.pallas_call(
    _scale1_body,
    grid=(NP // BM,),
    in_specs=[_spec_rows, _spec_w, _spec_nrm, _spec_nrm],
    out_specs=[_spec_cat, _spec_rows],
    out_shape=[jax.ShapeDtypeStruct((2, NP, HALF), jnp.float32),
               jax.ShapeDtypeStruct((NP, D), jnp.float32)],
)

_mid = pl.pallas_call(
    _mid_body,
    grid=(NP // BM,),
    in_specs=[_spec_cat, _spec_rows, _spec_nrm, _spec_nrm, _spec_b, _spec_w],
    out_specs=[_spec_cat, _spec_rows],
    out_shape=[jax.ShapeDtypeStruct((2, NP, HALF), jnp.float32),
               jax.ShapeDtypeStruct((NP, D), jnp.float32)],
)

_final = pl.pallas_call(
    _final_body,
    grid=(NP // BM,),
    in_specs=[_spec_cat, _spec_rows, _spec_nrm, _spec_nrm, _spec_b],
    out_specs=_spec_rows,
    out_shape=jax.ShapeDtypeStruct((N, D), jnp.float32),
)


@jax.jit
def kernel(x, edge_index, W1, b1, W2, b2):
    src = edge_index[0]
    dst = edge_index[1]
    npad = E_PAD - E
    padr = (jnp.arange(npad, dtype=jnp.int32) % (NP - N)) + N
    srcp = jnp.concatenate([src, padr])
    dstp = jnp.concatenate([dst, padr])
    src_deg = srcp.reshape(NCORE * NTILE, NCH_DEG, CHUNK)
    dst_deg = dstp.reshape(NCORE * NTILE, NCH_DEG, CHUNK)
    src_prop = srcp.reshape(NTILE, NCH_PROP, CHUNK)
    src_prop2 = jnp.stack([src_prop, src_prop + NP])
    dst_prop = dstp.reshape(NTILE, NCH_PROP, CHUNK)
    z1 = jnp.zeros((NP,), jnp.float32)
    z2 = jnp.zeros((NP, HALF), jnp.float32)

    cntp, loopp = _deg_call()(src_deg, dst_deg, z1)
    cnt3 = cntp.reshape(2, NP, 1)
    loop3 = loopp.reshape(2, NP, 1)

    y1cat, xw1 = _scale1(x, W1, cnt3, loop3)
    acc1 = _prop_call()(y1cat.reshape(2 * NP, HALF), src_prop2, dst_prop, z2)
    y2cat, hw2 = _mid(acc1, xw1, cnt3, loop3, b1.reshape(1, D), W2)
    acc2 = _prop_call()(y2cat.reshape(2 * NP, HALF), src_prop2, dst_prop, z2)
    return _final(acc2, hw2, cnt3, loop3, b2.reshape(1, D))


# speedup vs baseline: 12.8802x; 1.2284x over previous
import functools

import jax
import jax.numpy as jnp
from jax import lax
from jax.experimental import pallas as pl
from jax.experimental.pallas import tpu as pltpu
from jax.experimental.pallas import tpu_sc as plsc

N = 10000
D = 256
E = 160000
BM = 512
NP = 10240
NCORE = 2
NTILE = 16
RPT = NP // NTILE
CHUNK = 128
CHUNK_P = 64
E_PAD = 163840
NCH_DEG = E_PAD // (NCORE * NTILE * CHUNK)
NCH_PROP = E_PAD // (NTILE * CHUNK_P)
HALF = D // 2


@functools.cache
def _sc_mesh():
    return plsc.VectorSubcoreMesh(
        core_axis_name="c", subcore_axis_name="s",
        num_cores=NCORE, num_subcores=NTILE)


def _deg_body(src_hbm, dst_hbm, z1_hbm, cnt_hbm, loop_hbm,
              srcv, dstv, onesv, lbuf, cnt_sh, loop_sh):
    c = lax.axis_index("c")
    s = lax.axis_index("s")
    t = c * NTILE + s
    pltpu.sync_copy(src_hbm.at[t], srcv)
    pltpu.sync_copy(dst_hbm.at[t], dstv)
    r0 = s * RPT
    pltpu.sync_copy(z1_hbm.at[pl.ds(r0, RPT)], cnt_sh.at[pl.ds(r0, RPT)])
    pltpu.sync_copy(z1_hbm.at[pl.ds(r0, RPT)], loop_sh.at[pl.ds(r0, RPT)])
    for k in range(CHUNK // 16):
        onesv[pl.ds(k * 16, 16)] = jnp.full((16,), 1.0, jnp.float32)
    plsc.subcore_barrier()

    @pl.loop(0, NCH_DEG)
    def _chunk(j):
        for k in range(CHUNK // 16):
            sv = srcv[j, pl.ds(k * 16, 16)]
            dv = dstv[j, pl.ds(k * 16, 16)]
            lbuf[pl.ds(k * 16, 16)] = jnp.where(sv == dv, 1.0, 0.0)
        pltpu.sync_copy(onesv, cnt_sh.at[dstv.at[j]], add=True)
        pltpu.sync_copy(lbuf, loop_sh.at[dstv.at[j]], add=True)

    plsc.subcore_barrier()
    pltpu.sync_copy(cnt_sh.at[pl.ds(r0, RPT)], cnt_hbm.at[c, pl.ds(r0, RPT)])
    pltpu.sync_copy(loop_sh.at[pl.ds(r0, RPT)], loop_hbm.at[c, pl.ds(r0, RPT)])


@functools.cache
def _deg_call():
    return pl.kernel(
        _deg_body,
        out_type=[jax.ShapeDtypeStruct((NCORE, NP), jnp.float32),
                  jax.ShapeDtypeStruct((NCORE, NP), jnp.float32)],
        mesh=_sc_mesh(),
        scratch_types=[
            pltpu.VMEM((NCH_DEG, CHUNK), jnp.int32),
            pltpu.VMEM((NCH_DEG, CHUNK), jnp.int32),
            pltpu.VMEM((CHUNK,), jnp.float32),
            pltpu.VMEM((CHUNK,), jnp.float32),
            pltpu.VMEM_SHARED((NP,), jnp.float32),
            pltpu.VMEM_SHARED((NP,), jnp.float32),
        ],
    )


def _prop_body(ycat_hbm, src_hbm, dst_hbm, z2_hbm, out_hbm,
               idxv, dstv, buf0, buf1, sem0, sem1, acc_sh):
    c = lax.axis_index("c")
    s = lax.axis_index("s")
    pltpu.sync_copy(src_hbm.at[c, s], idxv)
    pltpu.sync_copy(dst_hbm.at[s], dstv)
    r0 = s * RPT
    pltpu.sync_copy(z2_hbm.at[pl.ds(r0, RPT)], acc_sh.at[pl.ds(r0, RPT)])
    plsc.subcore_barrier()

    def _start(row, half, buf, sem):
        pltpu.async_copy(
            ycat_hbm.at[idxv.at[row, pl.ds(half * CHUNK_P, CHUNK_P)]],
            buf, sem)

    def _wait(row, half, buf, sem):
        pltpu.make_async_copy(
            ycat_hbm.at[idxv.at[row, pl.ds(half * CHUNK_P, CHUNK_P)]],
            buf, sem).wait()

    _start(0, 0, buf0, sem0)

    @pl.loop(0, NCH_PROP // 2)
    def _pair(g):
        j0 = g * 2
        _start(g, 1, buf1, sem1)
        _wait(g, 0, buf0, sem0)
        pltpu.sync_copy(buf0, acc_sh.at[dstv.at[j0]], add=True)

        @pl.when(g < NCH_PROP // 2 - 1)
        def _():
            _start(g + 1, 0, buf0, sem0)

        _wait(g, 1, buf1, sem1)
        pltpu.sync_copy(buf1, acc_sh.at[dstv.at[j0 + 1]], add=True)

    plsc.subcore_barrier()
    pltpu.sync_copy(acc_sh.at[pl.ds(r0, RPT)], out_hbm.at[c, pl.ds(r0, RPT)])


@functools.cache
def _prop_call():
    return pl.kernel(
        _prop_body,
        out_type=jax.ShapeDtypeStruct((NCORE, NP, HALF), jnp.float32),
        mesh=_sc_mesh(),
        scratch_types=[
            pltpu.VMEM((NCH_PROP // 2, 2 * CHUNK_P), jnp.int32),
            pltpu.VMEM((NCH_PROP, CHUNK_P), jnp.int32),
            pltpu.VMEM((CHUNK_P, HALF), jnp.float32),
            pltpu.VMEM((CHUNK_P, HALF), jnp.float32),
            pltpu.SemaphoreType.DMA,
            pltpu.SemaphoreType.DMA,
            pltpu.VMEM_SHARED((NP, HALF), jnp.float32),
        ],
    )


def _norm(cnt_ref, loop_ref):
    cnt = cnt_ref[0] + cnt_ref[1]
    lc = loop_ref[0] + loop_ref[1]
    wl = jnp.where(lc == 0.0, 1.0, 0.0)
    d = lax.rsqrt(cnt + wl)
    return d, d * d * wl


def _scale1_body(x_ref, w_ref, cnt_ref, loop_ref, y_ref, xw_ref):
    xw = jnp.dot(x_ref[...], w_ref[...], preferred_element_type=jnp.float32)
    d, _ = _norm(cnt_ref, loop_ref)
    y = xw * d
    xw_ref[...] = xw
    y_ref[0] = y[:, :HALF]
    y_ref[1] = y[:, HALF:]


def _mid_body(acc_ref, xw_ref, cnt_ref, loop_ref, b_ref, w2_ref,
              y_ref, hw_ref):
    d, dw = _norm(cnt_ref, loop_ref)
    agg = jnp.concatenate([acc_ref[0], acc_ref[1]], axis=1)
    h = agg * d + xw_ref[...] * dw + b_ref[...]
    h = jnp.maximum(h, 0.0)
    hw = jnp.dot(h, w2_ref[...], preferred_element_type=jnp.float32)
    hw_ref[...] = hw
    y = hw * d
    y_ref[0] = y[:, :HALF]
    y_ref[1] = y[:, HALF:]


def _final_body(acc_ref, hw_ref, cnt_ref, loop_ref, b_ref, out_ref):
    d, dw = _norm(cnt_ref, loop_ref)
    agg = jnp.concatenate([acc_ref[0], acc_ref[1]], axis=1)
    o = agg * d + hw_ref[...] * dw + b_ref[...]
    m = jnp.max(o, axis=1, keepdims=True)
    e = jnp.exp(o - m)
    out_ref[...] = e / jnp.sum(e, axis=1, keepdims=True)


_spec_rows = pl.BlockSpec((BM, D), lambda i: (i, 0))
_spec_w = pl.BlockSpec((D, D), lambda i: (0, 0))
_spec_nrm = pl.BlockSpec((2, BM, 1), lambda i: (0, i, 0))
_spec_cat = pl.BlockSpec((2, BM, HALF), lambda i: (0, i, 0))
_spec_b = pl.BlockSpec((1, D), lambda i: (0, 0))

_scale1 = pl.pallas_call(
    _scale1_body,
    grid=(NP // BM,),
    in_specs=[_spec_rows, _spec_w, _spec_nrm, _spec_nrm],
    out_specs=[_spec_cat, _spec_rows],
    out_shape=[jax.ShapeDtypeStruct((2, NP, HALF), jnp.float32),
               jax.ShapeDtypeStruct((NP, D), jnp.float32)],
)

_mid = pl.pallas_call(
    _mid_body,
    grid=(NP // BM,),
    in_specs=[_spec_cat, _spec_rows, _spec_nrm, _spec_nrm, _spec_b, _spec_w],
    out_specs=[_spec_cat, _spec_rows],
    out_shape=[jax.ShapeDtypeStruct((2, NP, HALF), jnp.float32),
               jax.ShapeDtypeStruct((NP, D), jnp.float32)],
)

_final = pl.pallas_call(
    _final_body,
    grid=(NP // BM,),
    in_specs=[_spec_cat, _spec_rows, _spec_nrm, _spec_nrm, _spec_b],
    out_specs=_spec_rows,
    out_shape=jax.ShapeDtypeStruct((N, D), jnp.float32),
)


@jax.jit
def kernel(x, edge_index, W1, b1, W2, b2):
    src = edge_index[0]
    dst = edge_index[1]
    npad = E_PAD - E
    padr = (jnp.arange(npad, dtype=jnp.int32) % (NP - N)) + N
    srcp = jnp.concatenate([src, padr])
    dstp = jnp.concatenate([dst, padr])
    src_deg = srcp.reshape(NCORE * NTILE, NCH_DEG, CHUNK)
    dst_deg = dstp.reshape(NCORE * NTILE, NCH_DEG, CHUNK)
    src_prop = srcp.reshape(NTILE, NCH_PROP // 2, 2 * CHUNK_P)
    src_prop2 = jnp.stack([src_prop, src_prop + NP])
    dst_prop = dstp.reshape(NTILE, NCH_PROP, CHUNK_P)
    z1 = jnp.zeros((NP,), jnp.float32)
    z2 = jnp.zeros((NP, HALF), jnp.float32)

    cntp, loopp = _deg_call()(src_deg, dst_deg, z1)
    cnt3 = cntp.reshape(2, NP, 1)
    loop3 = loopp.reshape(2, NP, 1)

    y1cat, xw1 = _scale1(x, W1, cnt3, loop3)
    acc1 = _prop_call()(y1cat.reshape(2 * NP, HALF), src_prop2, dst_prop, z2)
    y2cat, hw2 = _mid(acc1, xw1, cnt3, loop3, b1.reshape(1, D), W2)
    acc2 = _prop_call()(y2cat.reshape(2 * NP, HALF), src_prop2, dst_prop, z2)
    return _final(acc2, hw2, cnt3, loop3, b2.reshape(1, D))


# speedup vs baseline: 12.9964x; 1.0090x over previous
import functools

import jax
import jax.numpy as jnp
from jax import lax
from jax.experimental import pallas as pl
from jax.experimental.pallas import tpu as pltpu
from jax.experimental.pallas import tpu_sc as plsc

N = 10000
D = 256
E = 160000
BM = 512
NP = 10240
NCORE = 2
NTILE = 16
RPT = NP // NTILE
CHUNK = 128
CHUNK_P = 64
E_PAD = 163840
NCH_DEG = E_PAD // (NCORE * NTILE * CHUNK)
NCH_PROP = E_PAD // (NTILE * CHUNK_P)
HALF = D // 2


@functools.cache
def _sc_mesh():
    return plsc.VectorSubcoreMesh(
        core_axis_name="c", subcore_axis_name="s",
        num_cores=NCORE, num_subcores=NTILE)


def _deg_body(src_hbm, dst_hbm, z1_hbm, cnt_hbm, loop_hbm,
              srcv, dstv, onesv, lbuf, cnt_sh, loop_sh):
    c = lax.axis_index("c")
    s = lax.axis_index("s")
    t = c * NTILE + s
    pltpu.sync_copy(src_hbm.at[t], srcv)
    pltpu.sync_copy(dst_hbm.at[t], dstv)
    r0 = s * RPT
    pltpu.sync_copy(z1_hbm.at[pl.ds(r0, RPT)], cnt_sh.at[pl.ds(r0, RPT)])
    pltpu.sync_copy(z1_hbm.at[pl.ds(r0, RPT)], loop_sh.at[pl.ds(r0, RPT)])
    for k in range(CHUNK // 16):
        onesv[pl.ds(k * 16, 16)] = jnp.full((16,), 1.0, jnp.float32)
    plsc.subcore_barrier()

    @pl.loop(0, NCH_DEG)
    def _chunk(j):
        for k in range(CHUNK // 16):
            sv = srcv[j, pl.ds(k * 16, 16)]
            dv = dstv[j, pl.ds(k * 16, 16)]
            lbuf[pl.ds(k * 16, 16)] = jnp.where(sv == dv, 1.0, 0.0)
        pltpu.sync_copy(onesv, cnt_sh.at[dstv.at[j]], add=True)
        pltpu.sync_copy(lbuf, loop_sh.at[dstv.at[j]], add=True)

    plsc.subcore_barrier()
    pltpu.sync_copy(cnt_sh.at[pl.ds(r0, RPT)], cnt_hbm.at[c, pl.ds(r0, RPT)])
    pltpu.sync_copy(loop_sh.at[pl.ds(r0, RPT)], loop_hbm.at[c, pl.ds(r0, RPT)])


@functools.cache
def _deg_call():
    return pl.kernel(
        _deg_body,
        out_type=[jax.ShapeDtypeStruct((NCORE, NP), jnp.float32),
                  jax.ShapeDtypeStruct((NCORE, NP), jnp.float32)],
        mesh=_sc_mesh(),
        scratch_types=[
            pltpu.VMEM((NCH_DEG, CHUNK), jnp.int32),
            pltpu.VMEM((NCH_DEG, CHUNK), jnp.int32),
            pltpu.VMEM((CHUNK,), jnp.float32),
            pltpu.VMEM((CHUNK,), jnp.float32),
            pltpu.VMEM_SHARED((NP,), jnp.float32),
            pltpu.VMEM_SHARED((NP,), jnp.float32),
        ],
    )


def _prop_body(y_hbm, src_hbm, dst_hbm, z2_hbm, out_hbm,
               idxv, dstv, buf0, buf1, sem0, sem1, acc_sh):
    c = lax.axis_index("c")
    s = lax.axis_index("s")
    pltpu.sync_copy(src_hbm.at[s], idxv)
    pltpu.sync_copy(dst_hbm.at[s], dstv)
    r0 = s * RPT
    pltpu.sync_copy(z2_hbm.at[pl.ds(r0, RPT)], acc_sh.at[pl.ds(r0, RPT)])
    plsc.subcore_barrier()

    coff = c * HALF

    def _start(row, half, buf, sem):
        pltpu.async_copy(
            y_hbm.at[idxv.at[row, pl.ds(half * CHUNK_P, CHUNK_P)],
                     pl.ds(coff, HALF)],
            buf, sem)

    def _wait(row, half, buf, sem):
        pltpu.make_async_copy(
            y_hbm.at[idxv.at[row, pl.ds(half * CHUNK_P, CHUNK_P)],
                     pl.ds(coff, HALF)],
            buf, sem).wait()

    _start(0, 0, buf0, sem0)

    @pl.loop(0, NCH_PROP // 2)
    def _pair(g):
        j0 = g * 2
        _start(g, 1, buf1, sem1)
        _wait(g, 0, buf0, sem0)
        pltpu.sync_copy(buf0, acc_sh.at[dstv.at[j0]], add=True)

        @pl.when(g < NCH_PROP // 2 - 1)
        def _():
            _start(g + 1, 0, buf0, sem0)

        _wait(g, 1, buf1, sem1)
        pltpu.sync_copy(buf1, acc_sh.at[dstv.at[j0 + 1]], add=True)

    plsc.subcore_barrier()
    pltpu.sync_copy(acc_sh.at[pl.ds(r0, RPT)], out_hbm.at[c, pl.ds(r0, RPT)])


@functools.cache
def _prop_call():
    return pl.kernel(
        _prop_body,
        out_type=jax.ShapeDtypeStruct((NCORE, NP, HALF), jnp.float32),
        mesh=_sc_mesh(),
        scratch_types=[
            pltpu.VMEM((NCH_PROP // 2, 2 * CHUNK_P), jnp.int32),
            pltpu.VMEM((NCH_PROP, CHUNK_P), jnp.int32),
            pltpu.VMEM((CHUNK_P, HALF), jnp.float32),
            pltpu.VMEM((CHUNK_P, HALF), jnp.float32),
            pltpu.SemaphoreType.DMA,
            pltpu.SemaphoreType.DMA,
            pltpu.VMEM_SHARED((NP, HALF), jnp.float32),
        ],
    )


def _norm(cnt_ref, loop_ref):
    cnt = cnt_ref[0] + cnt_ref[1]
    lc = loop_ref[0] + loop_ref[1]
    wl = jnp.where(lc == 0.0, 1.0, 0.0)
    deg = cnt + wl
    d = lax.rsqrt(deg)
    return d, d * d * wl, jnp.sqrt(deg)


def _scale1_body(x_ref, w_ref, cnt_ref, loop_ref, y_ref):
    xw = jnp.dot(x_ref[...], w_ref[...], preferred_element_type=jnp.float32)
    d, _, _ = _norm(cnt_ref, loop_ref)
    y_ref[...] = xw * d


def _mid_body(acc_ref, y1_ref, cnt_ref, loop_ref, b_ref, w2_ref, y_ref):
    d, dw, dinv = _norm(cnt_ref, loop_ref)
    agg = jnp.concatenate([acc_ref[0], acc_ref[1]], axis=1)
    xw = y1_ref[...] * dinv
    h = agg * d + xw * dw + b_ref[...]
    h = jnp.maximum(h, 0.0)
    hw = jnp.dot(h, w2_ref[...], preferred_element_type=jnp.float32)
    y_ref[...] = hw * d


def _final_body(acc_ref, y2_ref, cnt_ref, loop_ref, b_ref, out_ref):
    d, dw, dinv = _norm(cnt_ref, loop_ref)
    agg = jnp.concatenate([acc_ref[0], acc_ref[1]], axis=1)
    o = agg * d + (y2_ref[...] * dinv) * dw + b_ref[...]
    m = jnp.max(o, axis=1, keepdims=True)
    e = jnp.exp(o - m)
    out_ref[...] = e / jnp.sum(e, axis=1, keepdims=True)


_spec_rows = pl.BlockSpec((BM, D), lambda i: (i, 0))
_spec_w = pl.BlockSpec((D, D), lambda i: (0, 0))
_spec_nrm = pl.BlockSpec((2, BM, 1), lambda i: (0, i, 0))
_spec_cat = pl.BlockSpec((2, BM, HALF), lambda i: (0, i, 0))
_spec_b = pl.BlockSpec((1, D), lambda i: (0, 0))

_scale1 = pl.pallas_call(
    _scale1_body,
    grid=(NP // BM,),
    in_specs=[_spec_rows, _spec_w, _spec_nrm, _spec_nrm],
    out_specs=_spec_rows,
    out_shape=jax.ShapeDtypeStruct((NP, D), jnp.float32),
)

_mid = pl.pallas_call(
    _mid_body,
    grid=(NP // BM,),
    in_specs=[_spec_cat, _spec_rows, _spec_nrm, _spec_nrm, _spec_b, _spec_w],
    out_specs=_spec_rows,
    out_shape=jax.ShapeDtypeStruct((NP, D), jnp.float32),
)

_final = pl.pallas_call(
    _final_body,
    grid=(NP // BM,),
    in_specs=[_spec_cat, _spec_rows, _spec_nrm, _spec_nrm, _spec_b],
    out_specs=_spec_rows,
    out_shape=jax.ShapeDtypeStruct((N, D), jnp.float32),
)


@jax.jit
def kernel(x, edge_index, W1, b1, W2, b2):
    src = edge_index[0]
    dst = edge_index[1]
    npad = E_PAD - E
    padr = (jnp.arange(npad, dtype=jnp.int32) % (NP - N)) + N
    srcp = jnp.concatenate([src, padr])
    dstp = jnp.concatenate([dst, padr])
    src_deg = srcp.reshape(NCORE * NTILE, NCH_DEG, CHUNK)
    dst_deg = dstp.reshape(NCORE * NTILE, NCH_DEG, CHUNK)
    src_prop = srcp.reshape(NTILE, NCH_PROP // 2, 2 * CHUNK_P)
    dst_prop = dstp.reshape(NTILE, NCH_PROP, CHUNK_P)
    z1 = jnp.zeros((NP,), jnp.float32)
    z2 = jnp.zeros((NP, HALF), jnp.float32)

    cntp, loopp = _deg_call()(src_deg, dst_deg, z1)
    cnt3 = cntp.reshape(2, NP, 1)
    loop3 = loopp.reshape(2, NP, 1)

    y1 = _scale1(x, W1, cnt3, loop3)
    acc1 = _prop_call()(y1, src_prop, dst_prop, z2)
    y2 = _mid(acc1, y1, cnt3, loop3, b1.reshape(1, D), W2)
    acc2 = _prop_call()(y2, src_prop, dst_prop, z2)
    return _final(acc2, y2, cnt3, loop3, b2.reshape(1, D))
